# Initial kernel scaffold; baseline (speedup 1.0000x reference)
import jax
import jax.numpy as jnp
from jax.experimental import pallas as pl


def kernel(gene_x, cpg_x, mirna_x, gene_batch, cpg_batch, mirna_batch, mrna_W, mrna_b, cnv_W, cnv_b):
    raise NotImplementedError("write your pallas kernel here")


# speedup vs baseline: 8.8237x; 8.8237x over previous
import functools

import jax
import jax.numpy as jnp
from jax import lax
from jax.experimental import pallas as pl
from jax.experimental.pallas import tpu as pltpu
from jax.experimental.pallas import tpu_sc as plsc

_NSEG = 16
_H = 128
_NC = 2
_NS = 16
_NW = _NC * _NS
_C = 256

_N_GENE = 320000
_N_CPG = 400000
_N_MIRNA = 32000

_GENE_FULL = _N_GENE // _C
_CPG_FULL = _N_CPG // _C
_CPG_TAIL = _N_CPG - _CPG_FULL * _C
_MIRNA_FULL = _N_MIRNA // _C

_T_GENE = -(-_GENE_FULL // _NW)
_T_CPG = -(-_CPG_FULL // _NW)
_T_MIRNA = -(-_MIRNA_FULL // _NW)


def _sc_segment_sums(gene_x, gene_b, cpg_x, cpg_b, mirna_x, mirna_b,
                     pg, cg, pc, cc, pm, cm,
                     ids_buf, row_buf, acc_g, acc_c, acc_m,
                     cnt_g, cnt_c, cnt_m):
    wid = lax.axis_index("s") * _NC + lax.axis_index("c")
    z16 = jnp.zeros((16,), jnp.float32)
    iota = lax.iota(jnp.int32, 16)

    def zero_body(i, _):
        acc_g[pl.ds(i * 16, 16)] = z16
        acc_c[pl.ds(i * 16, 16)] = z16
        acc_m[pl.ds(i * 16, 16)] = z16
        return 0

    lax.fori_loop(0, (_NSEG * _H) // 16, zero_body, 0)
    cnt_g[...] = z16
    cnt_c[...] = z16
    cnt_m[...] = z16

    def process_chunk(x_hbm, b_hbm, base, nrows, acc, cnt):
        pltpu.sync_copy(b_hbm.at[pl.ds(base, nrows)], ids_buf.at[pl.ds(0, nrows)])
        pltpu.sync_copy(x_hbm.at[pl.ds(base, nrows)], row_buf.at[pl.ds(0, nrows)])
        first = ids_buf[pl.ds(0, 16)][0]
        last = ids_buf[pl.ds(nrows - 16, 16)][15]

        @pl.when(first == last)
        def _fast():
            def body(r, carry):
                return tuple(carry[c] + row_buf[r, pl.ds(16 * c, 16)]
                             for c in range(8))

            sums = lax.fori_loop(0, nrows, body, (z16,) * 8)
            off = first * _H
            for c in range(8):
                acc[pl.ds(off + 16 * c, 16)] = acc[pl.ds(off + 16 * c, 16)] + sums[c]
            cnt[...] = cnt[...] + jnp.where(iota == first,
                                            jnp.float32(nrows), 0.0)

        @pl.when(first != last)
        def _slow():
            def gbody(q, _):
                idvec = ids_buf[pl.ds(q * 16, 16)]
                for j in range(16):
                    seg = idvec[j]
                    off = seg * _H
                    r = q * 16 + j
                    for c in range(8):
                        acc[pl.ds(off + 16 * c, 16)] = (
                            acc[pl.ds(off + 16 * c, 16)]
                            + row_buf[r, pl.ds(16 * c, 16)])
                    cnt[...] = cnt[...] + jnp.where(iota == seg, 1.0, 0.0)
                return 0

            lax.fori_loop(0, nrows // 16, gbody, 0)

    def do_modality(x_hbm, b_hbm, acc, cnt, nfull, t_max):
        def tbody(t, _):
            g = wid + t * _NW

            @pl.when(g < nfull)
            def _():
                process_chunk(x_hbm, b_hbm, g * _C, _C, acc, cnt)

            return 0

        lax.fori_loop(0, t_max, tbody, 0)

    do_modality(gene_x, gene_b, acc_g, cnt_g, _GENE_FULL, _T_GENE)
    do_modality(cpg_x, cpg_b, acc_c, cnt_c, _CPG_FULL, _T_CPG)
    do_modality(mirna_x, mirna_b, acc_m, cnt_m, _MIRNA_FULL, _T_MIRNA)

    @pl.when(wid == _NW - 1)
    def _cpg_tail():
        process_chunk(cpg_x, cpg_b, _CPG_FULL * _C, _CPG_TAIL, acc_c, cnt_c)

    pltpu.sync_copy(acc_g, pg.at[wid])
    pltpu.sync_copy(cnt_g, cg.at[wid])
    pltpu.sync_copy(acc_c, pc.at[wid])
    pltpu.sync_copy(cnt_c, cc.at[wid])
    pltpu.sync_copy(acc_m, pm.at[wid])
    pltpu.sync_copy(cnt_m, cm.at[wid])


_sc_call = functools.partial(
    pl.kernel,
    out_type=[
        jax.ShapeDtypeStruct((_NW, _NSEG * _H), jnp.float32),
        jax.ShapeDtypeStruct((_NW, _NSEG), jnp.float32),
        jax.ShapeDtypeStruct((_NW, _NSEG * _H), jnp.float32),
        jax.ShapeDtypeStruct((_NW, _NSEG), jnp.float32),
        jax.ShapeDtypeStruct((_NW, _NSEG * _H), jnp.float32),
        jax.ShapeDtypeStruct((_NW, _NSEG), jnp.float32),
    ],
    mesh=plsc.VectorSubcoreMesh(core_axis_name="c", subcore_axis_name="s"),
    scratch_types=[
        pltpu.VMEM((_C,), jnp.int32),
        pltpu.VMEM((_C, _H), jnp.float32),
        pltpu.VMEM((_NSEG * _H,), jnp.float32),
        pltpu.VMEM((_NSEG * _H,), jnp.float32),
        pltpu.VMEM((_NSEG * _H,), jnp.float32),
        pltpu.VMEM((_NSEG,), jnp.float32),
        pltpu.VMEM((_NSEG,), jnp.float32),
        pltpu.VMEM((_NSEG,), jnp.float32),
    ],
)(_sc_segment_sums)


def _finalize(pg, cg, pc, cc, pm, cm, wm, bm, wc, bc,
              o_mrna, o_cnv, o_dna, o_mir):
    gs = jnp.sum(pg[...], axis=0)
    gc = jnp.sum(cg[...], axis=0)
    gmean = gs / jnp.maximum(gc, 1.0)
    gmask = gc > 0.0
    nt = (((1,), (1,)), ((), ()))
    mrna = lax.dot_general(gmean, wm[...], nt,
                           preferred_element_type=jnp.float32) + bm[...]
    o_mrna[...] = jnp.where(gmask, mrna, 0.0)
    cnv = lax.dot_general(gmean, wc[...], nt,
                          preferred_element_type=jnp.float32) + bc[...]
    o_cnv[...] = jnp.where(gmask, cnv, 0.0)
    o_dna[...] = jnp.sum(pc[...], axis=0) / jnp.maximum(jnp.sum(cc[...], axis=0), 1.0)
    o_mir[...] = jnp.sum(pm[...], axis=0) / jnp.maximum(jnp.sum(cm[...], axis=0), 1.0)


def kernel(gene_x, cpg_x, mirna_x, gene_batch, cpg_batch, mirna_batch,
           mrna_W, mrna_b, cnv_W, cnv_b):
    pg, cg, pc, cc, pm, cm = _sc_call(
        gene_x, gene_batch, cpg_x, cpg_batch, mirna_x, mirna_batch)

    outs = pl.pallas_call(
        _finalize,
        out_shape=[jax.ShapeDtypeStruct((_NSEG, _H), jnp.float32)] * 4,
    )(
        pg.reshape(_NW, _NSEG, _H), cg.reshape(_NW, _NSEG, 1),
        pc.reshape(_NW, _NSEG, _H), cc.reshape(_NW, _NSEG, 1),
        pm.reshape(_NW, _NSEG, _H), cm.reshape(_NW, _NSEG, 1),
        mrna_W, mrna_b.reshape(1, _H), cnv_W, cnv_b.reshape(1, _H),
    )
    return tuple(outs)


# speedup vs baseline: 15.1099x; 1.7124x over previous
import functools

import jax
import jax.numpy as jnp
from jax import lax
from jax.experimental import pallas as pl
from jax.experimental.pallas import tpu as pltpu
from jax.experimental.pallas import tpu_sc as plsc

_NSEG = 16
_H = 128
_NC = 2
_NS = 16
_NW = _NC * _NS
_C = 256

_N_GENE = 320000
_N_CPG = 400000
_N_MIRNA = 32000

_GENE_FULL = _N_GENE // _C
_CPG_FULL = _N_CPG // _C
_CPG_TAIL = _N_CPG - _CPG_FULL * _C
_MIRNA_FULL = _N_MIRNA // _C

_T_GENE = -(-_GENE_FULL // _NW)
_T_CPG = -(-_CPG_FULL // _NW)
_T_MIRNA = -(-_MIRNA_FULL // _NW)


def _sc_segment_sums(gene_x, gene_b, cpg_x, cpg_b, mirna_x, mirna_b,
                     pg, cg, pc, cc, pm, cm,
                     ids0, ids1, buf0, buf1, acc_g, acc_c, acc_m,
                     cnt_g, cnt_c, cnt_m,
                     sem_i0, sem_i1, sem_r0, sem_r1):
    wid = lax.axis_index("s") * _NC + lax.axis_index("c")
    z16 = jnp.zeros((16,), jnp.float32)
    iota = lax.iota(jnp.int32, 16)
    ids_bufs = (ids0, ids1)
    row_bufs = (buf0, buf1)
    sem_is = (sem_i0, sem_i1)
    sem_rs = (sem_r0, sem_r1)

    def zero_body(i, _):
        acc_g[pl.ds(i * 16, 16)] = z16
        acc_c[pl.ds(i * 16, 16)] = z16
        acc_m[pl.ds(i * 16, 16)] = z16
        return 0

    lax.fori_loop(0, (_NSEG * _H) // 16, zero_body, 0)
    cnt_g[...] = z16
    cnt_c[...] = z16
    cnt_m[...] = z16

    def process_buf(ids_buf, row_buf, nrows, acc, cnt):
        first = ids_buf[pl.ds(0, 16)][0]
        last = ids_buf[pl.ds(nrows - 16, 16)][15]

        @pl.when(first == last)
        def _fast():
            def body(r2, carry):
                r = r2 * 2
                mid = tuple(carry[c] + row_buf[r, pl.ds(16 * c, 16)]
                            for c in range(8))
                return tuple(mid[c] + row_buf[r + 1, pl.ds(16 * c, 16)]
                             for c in range(8))

            sums = lax.fori_loop(0, nrows // 2, body, (z16,) * 8)
            off = first * _H
            for c in range(8):
                acc[pl.ds(off + 16 * c, 16)] = acc[pl.ds(off + 16 * c, 16)] + sums[c]
            cnt[...] = cnt[...] + jnp.where(iota == first,
                                            jnp.float32(nrows), 0.0)

        @pl.when(first != last)
        def _slow():
            def gbody(q, _):
                idvec = ids_buf[pl.ds(q * 16, 16)]
                for j in range(16):
                    seg = idvec[j]
                    off = seg * _H
                    r = q * 16 + j
                    for c in range(8):
                        acc[pl.ds(off + 16 * c, 16)] = (
                            acc[pl.ds(off + 16 * c, 16)]
                            + row_buf[r, pl.ds(16 * c, 16)])
                    cnt[...] = cnt[...] + jnp.where(iota == seg, 1.0, 0.0)
                return 0

            lax.fori_loop(0, nrows // 16, gbody, 0)

    def process_chunk_sync(x_hbm, b_hbm, base, nrows, acc, cnt):
        pltpu.sync_copy(b_hbm.at[pl.ds(base, nrows)], ids0.at[pl.ds(0, nrows)])
        pltpu.sync_copy(x_hbm.at[pl.ds(base, nrows)], buf0.at[pl.ds(0, nrows)])
        process_buf(ids0, buf0, nrows, acc, cnt)

    def do_modality(x_hbm, b_hbm, acc, cnt, nfull, t_max):
        def start(g, b):
            pltpu.async_copy(b_hbm.at[pl.ds(g * _C, _C)], ids_bufs[b], sem_is[b])
            pltpu.async_copy(x_hbm.at[pl.ds(g * _C, _C)], row_bufs[b], sem_rs[b])

        def wait(g, b):
            pltpu.make_async_copy(
                b_hbm.at[pl.ds(g * _C, _C)], ids_bufs[b], sem_is[b]).wait()
            pltpu.make_async_copy(
                x_hbm.at[pl.ds(g * _C, _C)], row_bufs[b], sem_rs[b]).wait()

        start(wid, 0)
        start(wid + _NW, 1)

        def ubody(u, _):
            for b in range(2):
                t = u * 2 + b
                g = wid + t * _NW

                @pl.when(g < nfull)
                def _():
                    wait(g, b)
                    process_buf(ids_bufs[b], row_bufs[b], _C, acc, cnt)
                    g2 = g + 2 * _NW

                    @pl.when(g2 < nfull)
                    def _():
                        start(g2, b)

            return 0

        lax.fori_loop(0, -(-t_max // 2), ubody, 0)

    do_modality(gene_x, gene_b, acc_g, cnt_g, _GENE_FULL, _T_GENE)
    do_modality(cpg_x, cpg_b, acc_c, cnt_c, _CPG_FULL, _T_CPG)
    do_modality(mirna_x, mirna_b, acc_m, cnt_m, _MIRNA_FULL, _T_MIRNA)

    @pl.when(wid == _NW - 1)
    def _cpg_tail():
        process_chunk_sync(cpg_x, cpg_b, _CPG_FULL * _C, _CPG_TAIL,
                           acc_c, cnt_c)

    pltpu.sync_copy(acc_g, pg.at[wid])
    pltpu.sync_copy(cnt_g, cg.at[wid])
    pltpu.sync_copy(acc_c, pc.at[wid])
    pltpu.sync_copy(cnt_c, cc.at[wid])
    pltpu.sync_copy(acc_m, pm.at[wid])
    pltpu.sync_copy(cnt_m, cm.at[wid])


_sc_call = functools.partial(
    pl.kernel,
    out_type=[
        jax.ShapeDtypeStruct((_NW, _NSEG * _H), jnp.float32),
        jax.ShapeDtypeStruct((_NW, _NSEG), jnp.float32),
        jax.ShapeDtypeStruct((_NW, _NSEG * _H), jnp.float32),
        jax.ShapeDtypeStruct((_NW, _NSEG), jnp.float32),
        jax.ShapeDtypeStruct((_NW, _NSEG * _H), jnp.float32),
        jax.ShapeDtypeStruct((_NW, _NSEG), jnp.float32),
    ],
    mesh=plsc.VectorSubcoreMesh(core_axis_name="c", subcore_axis_name="s"),
    scratch_types=[
        pltpu.VMEM((_C,), jnp.int32),
        pltpu.VMEM((_C,), jnp.int32),
        pltpu.VMEM((_C, _H), jnp.float32),
        pltpu.VMEM((_C, _H), jnp.float32),
        pltpu.VMEM((_NSEG * _H,), jnp.float32),
        pltpu.VMEM((_NSEG * _H,), jnp.float32),
        pltpu.VMEM((_NSEG * _H,), jnp.float32),
        pltpu.VMEM((_NSEG,), jnp.float32),
        pltpu.VMEM((_NSEG,), jnp.float32),
        pltpu.VMEM((_NSEG,), jnp.float32),
        pltpu.SemaphoreType.DMA,
        pltpu.SemaphoreType.DMA,
        pltpu.SemaphoreType.DMA,
        pltpu.SemaphoreType.DMA,
    ],
)(_sc_segment_sums)


def _finalize(pg, cg, pc, cc, pm, cm, wm, bm, wc, bc,
              o_mrna, o_cnv, o_dna, o_mir):
    gs = jnp.sum(pg[...], axis=0)
    gc = jnp.sum(cg[...], axis=0)
    gmean = gs / jnp.maximum(gc, 1.0)
    gmask = gc > 0.0
    nt = (((1,), (1,)), ((), ()))
    mrna = lax.dot_general(gmean, wm[...], nt,
                           preferred_element_type=jnp.float32) + bm[...]
    o_mrna[...] = jnp.where(gmask, mrna, 0.0)
    cnv = lax.dot_general(gmean, wc[...], nt,
                          preferred_element_type=jnp.float32) + bc[...]
    o_cnv[...] = jnp.where(gmask, cnv, 0.0)
    o_dna[...] = jnp.sum(pc[...], axis=0) / jnp.maximum(jnp.sum(cc[...], axis=0), 1.0)
    o_mir[...] = jnp.sum(pm[...], axis=0) / jnp.maximum(jnp.sum(cm[...], axis=0), 1.0)


def kernel(gene_x, cpg_x, mirna_x, gene_batch, cpg_batch, mirna_batch,
           mrna_W, mrna_b, cnv_W, cnv_b):
    pg, cg, pc, cc, pm, cm = _sc_call(
        gene_x, gene_batch, cpg_x, cpg_batch, mirna_x, mirna_batch)

    outs = pl.pallas_call(
        _finalize,
        out_shape=[jax.ShapeDtypeStruct((_NSEG, _H), jnp.float32)] * 4,
    )(
        pg.reshape(_NW, _NSEG, _H), cg.reshape(_NW, _NSEG, 1),
        pc.reshape(_NW, _NSEG, _H), cc.reshape(_NW, _NSEG, 1),
        pm.reshape(_NW, _NSEG, _H), cm.reshape(_NW, _NSEG, 1),
        mrna_W, mrna_b.reshape(1, _H), cnv_W, cnv_b.reshape(1, _H),
    )
    return tuple(outs)


# speedup vs baseline: 22.1137x; 1.4635x over previous
import functools

import jax
import jax.numpy as jnp
from jax import lax
from jax.experimental import pallas as pl
from jax.experimental.pallas import tpu as pltpu
from jax.experimental.pallas import tpu_sc as plsc

_NSEG = 16
_H = 128
_NC = 2
_NS = 16
_NW = _NC * _NS
_C = 256

_N_GENE = 320000
_N_CPG = 400000
_N_MIRNA = 32000

_GENE_FULL = _N_GENE // _C
_MIRNA_FULL = _N_MIRNA // _C

_T_GENE = -(-_GENE_FULL // _NW)
_T_MIRNA = -(-_MIRNA_FULL // _NW)

_CPG_BLK = 8000
_CPG_NB = _N_CPG // _CPG_BLK


def _sc_segment_sums(gene_x, gene_b, mirna_x, mirna_b,
                     pg, cg, pm, cm,
                     ids0, ids1, buf0, buf1, acc_g, acc_m,
                     cnt_g, cnt_m,
                     sem_i0, sem_i1, sem_r0, sem_r1):
    wid = lax.axis_index("s") * _NC + lax.axis_index("c")
    z16 = jnp.zeros((16,), jnp.float32)
    iota = lax.iota(jnp.int32, 16)
    ids_bufs = (ids0, ids1)
    row_bufs = (buf0, buf1)
    sem_is = (sem_i0, sem_i1)
    sem_rs = (sem_r0, sem_r1)

    def zero_body(i, _):
        acc_g[pl.ds(i * 16, 16)] = z16
        acc_m[pl.ds(i * 16, 16)] = z16
        return 0

    lax.fori_loop(0, (_NSEG * _H) // 16, zero_body, 0)
    cnt_g[...] = z16
    cnt_m[...] = z16

    def process_buf(ids_buf, row_buf, nrows, acc, cnt):
        first = ids_buf[pl.ds(0, 16)][0]
        last = ids_buf[pl.ds(nrows - 16, 16)][15]

        @pl.when(first == last)
        def _fast():
            def body(r2, carry):
                r = r2 * 2
                mid = tuple(carry[c] + row_buf[r, pl.ds(16 * c, 16)]
                            for c in range(8))
                return tuple(mid[c] + row_buf[r + 1, pl.ds(16 * c, 16)]
                             for c in range(8))

            sums = lax.fori_loop(0, nrows // 2, body, (z16,) * 8)
            off = first * _H
            for c in range(8):
                acc[pl.ds(off + 16 * c, 16)] = acc[pl.ds(off + 16 * c, 16)] + sums[c]
            cnt[...] = cnt[...] + jnp.where(iota == first,
                                            jnp.float32(nrows), 0.0)

        @pl.when(first != last)
        def _slow():
            def gbody(q, _):
                idvec = ids_buf[pl.ds(q * 16, 16)]
                for j in range(16):
                    seg = idvec[j]
                    off = seg * _H
                    r = q * 16 + j
                    for c in range(8):
                        acc[pl.ds(off + 16 * c, 16)] = (
                            acc[pl.ds(off + 16 * c, 16)]
                            + row_buf[r, pl.ds(16 * c, 16)])
                    cnt[...] = cnt[...] + jnp.where(iota == seg, 1.0, 0.0)
                return 0

            lax.fori_loop(0, nrows // 16, gbody, 0)

    def do_modality(x_hbm, b_hbm, acc, cnt, nfull, t_max):
        def start(g, b):
            pltpu.async_copy(b_hbm.at[pl.ds(g * _C, _C)], ids_bufs[b], sem_is[b])
            pltpu.async_copy(x_hbm.at[pl.ds(g * _C, _C)], row_bufs[b], sem_rs[b])

        def wait(g, b):
            pltpu.make_async_copy(
                b_hbm.at[pl.ds(g * _C, _C)], ids_bufs[b], sem_is[b]).wait()
            pltpu.make_async_copy(
                x_hbm.at[pl.ds(g * _C, _C)], row_bufs[b], sem_rs[b]).wait()

        start(wid, 0)
        start(wid + _NW, 1)

        def ubody(u, _):
            for b in range(2):
                t = u * 2 + b
                g = wid + t * _NW

                @pl.when(g < nfull)
                def _():
                    wait(g, b)
                    process_buf(ids_bufs[b], row_bufs[b], _C, acc, cnt)
                    g2 = g + 2 * _NW

                    @pl.when(g2 < nfull)
                    def _():
                        start(g2, b)

            return 0

        lax.fori_loop(0, -(-t_max // 2), ubody, 0)

    do_modality(gene_x, gene_b, acc_g, cnt_g, _GENE_FULL, _T_GENE)
    do_modality(mirna_x, mirna_b, acc_m, cnt_m, _MIRNA_FULL, _T_MIRNA)

    pltpu.sync_copy(acc_g, pg.at[wid])
    pltpu.sync_copy(cnt_g, cg.at[wid])
    pltpu.sync_copy(acc_m, pm.at[wid])
    pltpu.sync_copy(cnt_m, cm.at[wid])


_sc_call = functools.partial(
    pl.kernel,
    out_type=[
        jax.ShapeDtypeStruct((_NW, _NSEG * _H), jnp.float32),
        jax.ShapeDtypeStruct((_NW, _NSEG), jnp.float32),
        jax.ShapeDtypeStruct((_NW, _NSEG * _H), jnp.float32),
        jax.ShapeDtypeStruct((_NW, _NSEG), jnp.float32),
    ],
    mesh=plsc.VectorSubcoreMesh(core_axis_name="c", subcore_axis_name="s"),
    scratch_types=[
        pltpu.VMEM((_C,), jnp.int32),
        pltpu.VMEM((_C,), jnp.int32),
        pltpu.VMEM((_C, _H), jnp.float32),
        pltpu.VMEM((_C, _H), jnp.float32),
        pltpu.VMEM((_NSEG * _H,), jnp.float32),
        pltpu.VMEM((_NSEG * _H,), jnp.float32),
        pltpu.VMEM((_NSEG,), jnp.float32),
        pltpu.VMEM((_NSEG,), jnp.float32),
        pltpu.SemaphoreType.DMA,
        pltpu.SemaphoreType.DMA,
        pltpu.SemaphoreType.DMA,
        pltpu.SemaphoreType.DMA,
    ],
)(_sc_segment_sums)


def _tc_cpg_body(ids_ref, x_ref, sum_ref, cnt_ref):
    @pl.when(pl.program_id(0) == 0)
    def _():
        sum_ref[...] = jnp.zeros_like(sum_ref)
        cnt_ref[...] = jnp.zeros_like(cnt_ref)

    ids = ids_ref[0, 0, :]
    onehot = (ids[:, None]
              == lax.broadcasted_iota(jnp.int32, (_CPG_BLK, _NSEG), 1)
              ).astype(jnp.float32)
    psum = lax.dot_general(onehot, x_ref[...], (((0,), (0,)), ((), ())),
                           preferred_element_type=jnp.float32)
    sum_ref[...] += psum
    cnt_ref[...] += jnp.sum(onehot, axis=0, keepdims=True)


_tc_cpg = pl.pallas_call(
    _tc_cpg_body,
    grid=(_CPG_NB,),
    in_specs=[
        pl.BlockSpec((1, 1, _CPG_BLK), lambda i: (i, 0, 0)),
        pl.BlockSpec((_CPG_BLK, _H), lambda i: (i, 0)),
    ],
    out_specs=[
        pl.BlockSpec((_NSEG, _H), lambda i: (0, 0)),
        pl.BlockSpec((1, _NSEG), lambda i: (0, 0)),
    ],
    out_shape=[
        jax.ShapeDtypeStruct((_NSEG, _H), jnp.float32),
        jax.ShapeDtypeStruct((1, _NSEG), jnp.float32),
    ],
)


def _finalize(pg, cg, ds_, dc, pm, cm, wm, bm, wc, bc,
              o_mrna, o_cnv, o_dna, o_mir):
    gs = jnp.sum(pg[...], axis=0)
    gc = jnp.sum(cg[...], axis=0)
    gmean = gs / jnp.maximum(gc, 1.0)
    gmask = gc > 0.0
    nt = (((1,), (1,)), ((), ()))
    mrna = lax.dot_general(gmean, wm[...], nt,
                           preferred_element_type=jnp.float32) + bm[...]
    o_mrna[...] = jnp.where(gmask, mrna, 0.0)
    cnv = lax.dot_general(gmean, wc[...], nt,
                          preferred_element_type=jnp.float32) + bc[...]
    o_cnv[...] = jnp.where(gmask, cnv, 0.0)
    o_dna[...] = ds_[...] / jnp.maximum(dc[...], 1.0)
    o_mir[...] = jnp.sum(pm[...], axis=0) / jnp.maximum(jnp.sum(cm[...], axis=0), 1.0)


def kernel(gene_x, cpg_x, mirna_x, gene_batch, cpg_batch, mirna_batch,
           mrna_W, mrna_b, cnv_W, cnv_b):
    pg, cg, pm, cm = _sc_call(gene_x, gene_batch, mirna_x, mirna_batch)
    cpg_sum, cpg_cnt = _tc_cpg(
        cpg_batch.reshape(_CPG_NB, 1, _CPG_BLK), cpg_x)

    outs = pl.pallas_call(
        _finalize,
        out_shape=[jax.ShapeDtypeStruct((_NSEG, _H), jnp.float32)] * 4,
    )(
        pg.reshape(_NW, _NSEG, _H), cg.reshape(_NW, _NSEG, 1),
        cpg_sum, cpg_cnt.reshape(_NSEG, 1),
        pm.reshape(_NW, _NSEG, _H), cm.reshape(_NW, _NSEG, 1),
        mrna_W, mrna_b.reshape(1, _H), cnv_W, cnv_b.reshape(1, _H),
    )
    return tuple(outs)


# speedup vs baseline: 22.3151x; 1.0091x over previous
import functools

import jax
import jax.numpy as jnp
from jax import lax
from jax.experimental import pallas as pl
from jax.experimental.pallas import tpu as pltpu
from jax.experimental.pallas import tpu_sc as plsc

_NSEG = 16
_H = 128
_NC = 2
_NS = 16
_NW = _NC * _NS
_C = 256

_N_GENE = 320000
_N_CPG = 400000
_N_MIRNA = 32000

_GENE_FULL = _N_GENE // _C
_MIRNA_FULL = _N_MIRNA // _C
_ALL_FULL = _GENE_FULL + _MIRNA_FULL
_T_ALL = -(-_ALL_FULL // _NW)

_BLK = 16384
_CPG_NB = -(-_N_CPG // _BLK)


def _sc_segment_sums(gene_x, gene_b, mirna_x, mirna_b,
                     pg, cg, pm, cm,
                     ids0, ids1, buf0, buf1, acc_g, acc_m,
                     cnt_g, cnt_m,
                     sem_i0, sem_i1, sem_r0, sem_r1):
    wid = lax.axis_index("s") * _NC + lax.axis_index("c")
    z16 = jnp.zeros((16,), jnp.float32)
    iota = lax.iota(jnp.int32, 16)
    ids_bufs = (ids0, ids1)
    row_bufs = (buf0, buf1)
    sem_is = (sem_i0, sem_i1)
    sem_rs = (sem_r0, sem_r1)

    def zero_body(s, _):
        for c in range(8):
            acc_g[s, pl.ds(16 * c, 16)] = z16
            acc_m[s, pl.ds(16 * c, 16)] = z16
        return 0

    lax.fori_loop(0, _NSEG, zero_body, 0)
    cnt_g[...] = z16
    cnt_m[...] = z16

    def process_buf(ids_buf, row_buf, nrows, acc, cnt):
        first = ids_buf[pl.ds(0, 16)][0]
        last = ids_buf[pl.ds(nrows - 16, 16)][15]

        @pl.when(first == last)
        def _fast():
            def body(r2, carry):
                r = r2 * 2
                mid = tuple(carry[c] + row_buf[r, pl.ds(16 * c, 16)]
                            for c in range(8))
                return tuple(mid[c] + row_buf[r + 1, pl.ds(16 * c, 16)]
                             for c in range(8))

            sums = lax.fori_loop(0, nrows // 2, body, (z16,) * 8)
            for c in range(8):
                acc[first, pl.ds(16 * c, 16)] = (
                    acc[first, pl.ds(16 * c, 16)] + sums[c])
            cnt[...] = cnt[...] + jnp.where(iota == first,
                                            jnp.float32(nrows), 0.0)

        @pl.when(first != last)
        def _slow():
            def gbody(q, _):
                idvec = ids_buf[pl.ds(q * 16, 16)]
                for j in range(16):
                    seg = idvec[j]
                    r = q * 16 + j
                    for c in range(8):
                        acc[seg, pl.ds(16 * c, 16)] = (
                            acc[seg, pl.ds(16 * c, 16)]
                            + row_buf[r, pl.ds(16 * c, 16)])
                    cnt[...] = cnt[...] + jnp.where(iota == seg, 1.0, 0.0)
                return 0

            lax.fori_loop(0, nrows // 16, gbody, 0)

    def start(g, b):
        @pl.when(g < _GENE_FULL)
        def _():
            pltpu.async_copy(gene_b.at[pl.ds(g * _C, _C)],
                             ids_bufs[b], sem_is[b])
            pltpu.async_copy(gene_x.at[pl.ds(g * _C, _C)],
                             row_bufs[b], sem_rs[b])

        @pl.when(g >= _GENE_FULL)
        def _():
            m = g - _GENE_FULL
            pltpu.async_copy(mirna_b.at[pl.ds(m * _C, _C)],
                             ids_bufs[b], sem_is[b])
            pltpu.async_copy(mirna_x.at[pl.ds(m * _C, _C)],
                             row_bufs[b], sem_rs[b])

    def wait(b):
        pltpu.make_async_copy(
            gene_b.at[pl.ds(0, _C)], ids_bufs[b], sem_is[b]).wait()
        pltpu.make_async_copy(
            gene_x.at[pl.ds(0, _C)], row_bufs[b], sem_rs[b]).wait()

    start(wid, 0)
    start(wid + _NW, 1)

    def ubody(u, _):
        for b in range(2):
            t = u * 2 + b
            g = wid + t * _NW

            @pl.when(g < _ALL_FULL)
            def _():
                wait(b)

                @pl.when(g < _GENE_FULL)
                def _():
                    process_buf(ids_bufs[b], row_bufs[b], _C, acc_g, cnt_g)

                @pl.when(g >= _GENE_FULL)
                def _():
                    process_buf(ids_bufs[b], row_bufs[b], _C, acc_m, cnt_m)

                g2 = g + 2 * _NW

                @pl.when(g2 < _ALL_FULL)
                def _():
                    start(g2, b)

        return 0

    lax.fori_loop(0, -(-_T_ALL // 2), ubody, 0)

    pltpu.sync_copy(acc_g, pg.at[wid])
    pltpu.sync_copy(cnt_g, cg.at[wid])
    pltpu.sync_copy(acc_m, pm.at[wid])
    pltpu.sync_copy(cnt_m, cm.at[wid])


_sc_call = functools.partial(
    pl.kernel,
    out_type=[
        jax.ShapeDtypeStruct((_NW, _NSEG, _H), jnp.float32),
        jax.ShapeDtypeStruct((_NW, _NSEG), jnp.float32),
        jax.ShapeDtypeStruct((_NW, _NSEG, _H), jnp.float32),
        jax.ShapeDtypeStruct((_NW, _NSEG), jnp.float32),
    ],
    mesh=plsc.VectorSubcoreMesh(core_axis_name="c", subcore_axis_name="s"),
    scratch_types=[
        pltpu.VMEM((_C,), jnp.int32),
        pltpu.VMEM((_C,), jnp.int32),
        pltpu.VMEM((_C, _H), jnp.float32),
        pltpu.VMEM((_C, _H), jnp.float32),
        pltpu.VMEM((_NSEG, _H), jnp.float32),
        pltpu.VMEM((_NSEG, _H), jnp.float32),
        pltpu.VMEM((_NSEG,), jnp.float32),
        pltpu.VMEM((_NSEG,), jnp.float32),
        pltpu.SemaphoreType.DMA,
        pltpu.SemaphoreType.DMA,
        pltpu.SemaphoreType.DMA,
        pltpu.SemaphoreType.DMA,
    ],
)(_sc_segment_sums)


def _tc_onehot_body(ids_ref, x_ref, sum_ref, cnt_ref):
    @pl.when(pl.program_id(0) == 0)
    def _():
        sum_ref[...] = jnp.zeros_like(sum_ref)
        cnt_ref[...] = jnp.zeros_like(cnt_ref)

    ids = ids_ref[...]
    valid = (pl.program_id(0) * _BLK
             + lax.broadcasted_iota(jnp.int32, (_BLK, _NSEG), 0)) < _N_CPG
    onehot = ((ids[:, None]
               == lax.broadcasted_iota(jnp.int32, (_BLK, _NSEG), 1))
              & valid).astype(jnp.float32)
    psum = lax.dot_general(onehot, x_ref[...], (((0,), (0,)), ((), ())),
                           preferred_element_type=jnp.float32)
    sum_ref[...] += psum
    cnt_ref[...] += jnp.sum(onehot, axis=0, keepdims=True)


_tc_onehot = pl.pallas_call(
    _tc_onehot_body,
    grid=(_CPG_NB,),
    in_specs=[
        pl.BlockSpec((_BLK,), lambda i: (i,)),
        pl.BlockSpec((_BLK, _H), lambda i: (i, 0)),
    ],
    out_specs=[
        pl.BlockSpec((_NSEG, _H), lambda i: (0, 0)),
        pl.BlockSpec((1, _NSEG), lambda i: (0, 0)),
    ],
    out_shape=[
        jax.ShapeDtypeStruct((_NSEG, _H), jnp.float32),
        jax.ShapeDtypeStruct((1, _NSEG), jnp.float32),
    ],
)


def _finalize(pg, cg, ds_, dc, pm, cm, wm, bm, wc, bc,
              o_mrna, o_cnv, o_dna, o_mir):
    gs = jnp.sum(pg[...], axis=0)
    gc = jnp.sum(cg[...], axis=0, keepdims=True).T
    gmean = gs / jnp.maximum(gc, 1.0)
    gmask = gc > 0.0
    nt = (((1,), (1,)), ((), ()))
    mrna = lax.dot_general(gmean, wm[...], nt,
                           preferred_element_type=jnp.float32) + bm[...]
    o_mrna[...] = jnp.where(gmask, mrna, 0.0)
    cnv = lax.dot_general(gmean, wc[...], nt,
                          preferred_element_type=jnp.float32) + bc[...]
    o_cnv[...] = jnp.where(gmask, cnv, 0.0)
    o_dna[...] = ds_[...] / jnp.maximum(dc[...].T, 1.0)
    mc2 = jnp.sum(cm[...], axis=0, keepdims=True).T
    o_mir[...] = jnp.sum(pm[...], axis=0) / jnp.maximum(mc2, 1.0)


def kernel(gene_x, cpg_x, mirna_x, gene_batch, cpg_batch, mirna_batch,
           mrna_W, mrna_b, cnv_W, cnv_b):
    pg, cg, pm, cm = _sc_call(gene_x, gene_batch, mirna_x, mirna_batch)
    cpg_sum, cpg_cnt = _tc_onehot(cpg_batch, cpg_x)

    outs = pl.pallas_call(
        _finalize,
        out_shape=[jax.ShapeDtypeStruct((_NSEG, _H), jnp.float32)] * 4,
    )(
        pg, cg, cpg_sum, cpg_cnt, pm, cm,
        mrna_W, mrna_b.reshape(1, _H), cnv_W, cnv_b.reshape(1, _H),
    )
    return tuple(outs)


# speedup vs baseline: 23.5646x; 1.0560x over previous
import functools

import jax
import jax.numpy as jnp
from jax import lax
from jax.experimental import pallas as pl
from jax.experimental.pallas import tpu as pltpu
from jax.experimental.pallas import tpu_sc as plsc

_NSEG = 16
_H = 128
_NC = 2
_NS = 16
_NW = _NC * _NS
_C = 256

_N_GENE = 320000
_N_CPG = 400000
_N_MIRNA = 32000

_GENE_FULL = _N_GENE // _C
_MIRNA_FULL = _N_MIRNA // _C
_ALL_FULL = _GENE_FULL + _MIRNA_FULL
_T_ALL = -(-_ALL_FULL // _NW)

_BLK = 8192
_CPG_NB = -(-_N_CPG // _BLK)


def _sc_segment_sums(gene_x, gene_b, mirna_x, mirna_b,
                     pg, cg, pm, cm,
                     ids0, ids1, buf0, buf1, acc_g, acc_m,
                     cnt_g, cnt_m,
                     sem_i0, sem_i1, sem_r0, sem_r1):
    wid = lax.axis_index("s") * _NC + lax.axis_index("c")
    z16 = jnp.zeros((16,), jnp.float32)
    iota = lax.iota(jnp.int32, 16)
    ids_bufs = (ids0, ids1)
    row_bufs = (buf0, buf1)
    sem_is = (sem_i0, sem_i1)
    sem_rs = (sem_r0, sem_r1)

    def zero_body(s, _):
        for c in range(8):
            acc_g[s, pl.ds(16 * c, 16)] = z16
            acc_m[s, pl.ds(16 * c, 16)] = z16
        return 0

    lax.fori_loop(0, _NSEG, zero_body, 0)
    cnt_g[...] = z16
    cnt_m[...] = z16

    def process_buf(ids_buf, row_buf, nrows, acc, cnt):
        first = ids_buf[pl.ds(0, 16)][0]
        last = ids_buf[pl.ds(nrows - 16, 16)][15]

        @pl.when(first == last)
        def _fast():
            def body(r2, carry):
                r = r2 * 2
                mid = tuple(carry[c] + row_buf[r, pl.ds(16 * c, 16)]
                            for c in range(8))
                return tuple(mid[c] + row_buf[r + 1, pl.ds(16 * c, 16)]
                             for c in range(8))

            sums = lax.fori_loop(0, nrows // 2, body, (z16,) * 8)
            for c in range(8):
                acc[first, pl.ds(16 * c, 16)] = (
                    acc[first, pl.ds(16 * c, 16)] + sums[c])
            cnt[...] = cnt[...] + jnp.where(iota == first,
                                            jnp.float32(nrows), 0.0)

        @pl.when(first != last)
        def _slow():
            def gbody(q, _):
                idvec = ids_buf[pl.ds(q * 16, 16)]
                for j in range(16):
                    seg = idvec[j]
                    r = q * 16 + j
                    for c in range(8):
                        acc[seg, pl.ds(16 * c, 16)] = (
                            acc[seg, pl.ds(16 * c, 16)]
                            + row_buf[r, pl.ds(16 * c, 16)])
                    cnt[...] = cnt[...] + jnp.where(iota == seg, 1.0, 0.0)
                return 0

            lax.fori_loop(0, nrows // 16, gbody, 0)

    def start(g, b):
        @pl.when(g < _GENE_FULL)
        def _():
            pltpu.async_copy(gene_b.at[pl.ds(g * _C, _C)],
                             ids_bufs[b], sem_is[b])
            pltpu.async_copy(gene_x.at[pl.ds(g * _C, _C)],
                             row_bufs[b], sem_rs[b])

        @pl.when(g >= _GENE_FULL)
        def _():
            m = g - _GENE_FULL
            pltpu.async_copy(mirna_b.at[pl.ds(m * _C, _C)],
                             ids_bufs[b], sem_is[b])
            pltpu.async_copy(mirna_x.at[pl.ds(m * _C, _C)],
                             row_bufs[b], sem_rs[b])

    def wait(b):
        pltpu.make_async_copy(
            gene_b.at[pl.ds(0, _C)], ids_bufs[b], sem_is[b]).wait()
        pltpu.make_async_copy(
            gene_x.at[pl.ds(0, _C)], row_bufs[b], sem_rs[b]).wait()

    start(wid, 0)
    start(wid + _NW, 1)

    def ubody(u, _):
        for b in range(2):
            t = u * 2 + b
            g = wid + t * _NW

            @pl.when(g < _ALL_FULL)
            def _():
                wait(b)

                @pl.when(g < _GENE_FULL)
                def _():
                    process_buf(ids_bufs[b], row_bufs[b], _C, acc_g, cnt_g)

                @pl.when(g >= _GENE_FULL)
                def _():
                    process_buf(ids_bufs[b], row_bufs[b], _C, acc_m, cnt_m)

                g2 = g + 2 * _NW

                @pl.when(g2 < _ALL_FULL)
                def _():
                    start(g2, b)

        return 0

    lax.fori_loop(0, -(-_T_ALL // 2), ubody, 0)

    pltpu.sync_copy(acc_g, pg.at[wid])
    pltpu.sync_copy(cnt_g, cg.at[wid])
    pltpu.sync_copy(acc_m, pm.at[wid])
    pltpu.sync_copy(cnt_m, cm.at[wid])


_sc_call = functools.partial(
    pl.kernel,
    out_type=[
        jax.ShapeDtypeStruct((_NW, _NSEG, _H), jnp.float32),
        jax.ShapeDtypeStruct((_NW, _NSEG), jnp.float32),
        jax.ShapeDtypeStruct((_NW, _NSEG, _H), jnp.float32),
        jax.ShapeDtypeStruct((_NW, _NSEG), jnp.float32),
    ],
    mesh=plsc.VectorSubcoreMesh(core_axis_name="c", subcore_axis_name="s"),
    scratch_types=[
        pltpu.VMEM((_C,), jnp.int32),
        pltpu.VMEM((_C,), jnp.int32),
        pltpu.VMEM((_C, _H), jnp.float32),
        pltpu.VMEM((_C, _H), jnp.float32),
        pltpu.VMEM((_NSEG, _H), jnp.float32),
        pltpu.VMEM((_NSEG, _H), jnp.float32),
        pltpu.VMEM((_NSEG,), jnp.float32),
        pltpu.VMEM((_NSEG,), jnp.float32),
        pltpu.SemaphoreType.DMA,
        pltpu.SemaphoreType.DMA,
        pltpu.SemaphoreType.DMA,
        pltpu.SemaphoreType.DMA,
    ],
)(_sc_segment_sums)


def _tc_onehot_body(ids_ref, x_ref, sum_ref, cnt_ref):
    @pl.when(pl.program_id(0) == 0)
    def _():
        sum_ref[...] = jnp.zeros_like(sum_ref)
        cnt_ref[...] = jnp.zeros_like(cnt_ref)

    ids = ids_ref[...]
    valid = (pl.program_id(0) * _BLK
             + lax.broadcasted_iota(jnp.int32, (_BLK, _NSEG), 0)) < _N_CPG
    onehot = ((ids[:, None]
               == lax.broadcasted_iota(jnp.int32, (_BLK, _NSEG), 1))
              & valid).astype(jnp.float32)
    psum = lax.dot_general(onehot, x_ref[...], (((0,), (0,)), ((), ())),
                           preferred_element_type=jnp.float32)
    sum_ref[...] += psum
    cnt_ref[...] += jnp.sum(onehot, axis=0, keepdims=True)


_tc_onehot = pl.pallas_call(
    _tc_onehot_body,
    grid=(_CPG_NB,),
    in_specs=[
        pl.BlockSpec((_BLK,), lambda i: (i,)),
        pl.BlockSpec((_BLK, _H), lambda i: (i, 0)),
    ],
    out_specs=[
        pl.BlockSpec((_NSEG, _H), lambda i: (0, 0)),
        pl.BlockSpec((1, _NSEG), lambda i: (0, 0)),
    ],
    out_shape=[
        jax.ShapeDtypeStruct((_NSEG, _H), jnp.float32),
        jax.ShapeDtypeStruct((1, _NSEG), jnp.float32),
    ],
)


def _finalize(pg, cg, ds_, dc, pm, cm, wm, bm, wc, bc,
              o_mrna, o_cnv, o_dna, o_mir):
    gs = jnp.sum(pg[...], axis=0)
    gc = jnp.sum(cg[...], axis=0, keepdims=True).T
    gmean = gs / jnp.maximum(gc, 1.0)
    gmask = gc > 0.0
    nt = (((1,), (1,)), ((), ()))
    mrna = lax.dot_general(gmean, wm[...], nt,
                           preferred_element_type=jnp.float32) + bm[...]
    o_mrna[...] = jnp.where(gmask, mrna, 0.0)
    cnv = lax.dot_general(gmean, wc[...], nt,
                          preferred_element_type=jnp.float32) + bc[...]
    o_cnv[...] = jnp.where(gmask, cnv, 0.0)
    o_dna[...] = ds_[...] / jnp.maximum(dc[...].T, 1.0)
    mc2 = jnp.sum(cm[...], axis=0, keepdims=True).T
    o_mir[...] = jnp.sum(pm[...], axis=0) / jnp.maximum(mc2, 1.0)


def kernel(gene_x, cpg_x, mirna_x, gene_batch, cpg_batch, mirna_batch,
           mrna_W, mrna_b, cnv_W, cnv_b):
    pg, cg, pm, cm = _sc_call(gene_x, gene_batch, mirna_x, mirna_batch)
    cpg_sum, cpg_cnt = _tc_onehot(cpg_batch, cpg_x)

    outs = pl.pallas_call(
        _finalize,
        out_shape=[jax.ShapeDtypeStruct((_NSEG, _H), jnp.float32)] * 4,
    )(
        pg, cg, cpg_sum, cpg_cnt, pm, cm,
        mrna_W, mrna_b.reshape(1, _H), cnv_W, cnv_b.reshape(1, _H),
    )
    return tuple(outs)
